# baseline (speedup 1.0000x reference)
import jax
import jax.numpy as jnp
from jax.experimental import pallas as pl
from jax.experimental.pallas import tpu as pltpu

_N = 1_000_000
_NCHUNK = 6
_BIG = 999_936 // _NCHUNK
_TAIL_OFF = _NCHUNK * _BIG
_TAIL = _N - _TAIL_OFF
_OFFS = tuple(i * _BIG for i in range(_NCHUNK)) + (_TAIL_OFF,)
_SIZES = (_BIG,) * _NCHUNK + (_TAIL,)
_NPIECE = _NCHUNK + 1


def _copy_body(in_hbm, out_hbm, *rest):
    bufs = rest[:_NPIECE]
    in_sem, out_sem = rest[_NPIECE], rest[_NPIECE + 1]
    for i in range(_NPIECE):
        pltpu.make_async_copy(
            in_hbm.at[pl.ds(_OFFS[i], _SIZES[i])], bufs[i], in_sem.at[i]
        ).start()
    for i in range(_NPIECE):
        pltpu.make_async_copy(
            in_hbm.at[pl.ds(_OFFS[i], _SIZES[i])], bufs[i], in_sem.at[i]
        ).wait()
        pltpu.make_async_copy(
            bufs[i], out_hbm.at[pl.ds(_OFFS[i], _SIZES[i])], out_sem.at[i]
        ).start()
    for i in range(_NPIECE):
        pltpu.make_async_copy(
            bufs[i], out_hbm.at[pl.ds(_OFFS[i], _SIZES[i])], out_sem.at[i]
        ).wait()


def kernel(goal_logits):
    return pl.pallas_call(
        _copy_body,
        out_shape=jax.ShapeDtypeStruct((_N,), jnp.float32),
        in_specs=[pl.BlockSpec(memory_space=pl.ANY)],
        out_specs=pl.BlockSpec(memory_space=pl.ANY),
        scratch_shapes=(
            [pltpu.VMEM((s,), jnp.float32) for s in _SIZES]
            + [pltpu.SemaphoreType.DMA((_NPIECE,)),
               pltpu.SemaphoreType.DMA((_NPIECE,))]
        ),
    )(goal_logits)
